# baseline (speedup 1.0000x reference)
import functools

import jax
import jax.numpy as jnp
from jax import lax
from jax.experimental import pallas as pl
from jax.experimental.pallas import tpu as pltpu
from jax.experimental.pallas import tpu_sc as plsc

N, E, G, H = 50000, 800000, 64, 128
BLK = 1000

R = 6272
NQ = 8
ACC_ROWS = 6400
DUMP = R
E_PAD = 800256
EPT = E_PAD // 32
CH_V = 521
CPT = 3456
CAP = 32 * CPT
SPT = CAP // 16
NBLK = SPT // 128
CPR = R // 16
ZR = 16

_mesh = plsc.VectorSubcoreMesh(core_axis_name="c", subcore_axis_name="s")


def _lrelu(x):
    return jnp.where(x >= 0, x, 0.01 * x)


@functools.partial(
    pl.kernel,
    out_type=(jax.ShapeDtypeStruct((NQ * CAP,), jnp.int32),
              jax.ShapeDtypeStruct((NQ * CAP,), jnp.float32),
              jax.ShapeDtypeStruct((NQ * CAP,), jnp.int32)),
    mesh=_mesh,
    compiler_params=pltpu.CompilerParams(needs_layout_passes=False),
    scratch_types=[
        pltpu.VMEM((CH_V * 16,), jnp.int32),
        pltpu.VMEM((CH_V * 16,), jnp.int32),
        pltpu.VMEM((CH_V * 16,), jnp.float32),
        pltpu.VMEM((NQ, CPT), jnp.int32),
        pltpu.VMEM((NQ, CPT), jnp.float32),
        pltpu.VMEM((NQ, CPT), jnp.int32),
    ],
)
def _bucket_k(src_hbm, dst_hbm, ea_hbm, bsrc_hbm, bea_hbm, brel_hbm,
              csrc, cdst, cea, ssrc, sea, srel):
    c = lax.axis_index("c")
    s = lax.axis_index("s")
    pt = c * 16 + s
    iota = lax.iota(jnp.int32, 16)

    def fill(v, carry):
        base = v * 16
        spread = (base + iota) & 1023
        rel = DUMP + ((base + iota) & 63)
        zero = jnp.zeros((16,), jnp.float32)
        for b in range(NQ):
            ssrc[b, pl.ds(base, 16)] = spread
            sea[b, pl.ds(base, 16)] = zero
            srel[b, pl.ds(base, 16)] = rel
        return carry
    lax.fori_loop(0, CPT // 16, fill, 0)

    offs = tuple(jnp.zeros((16,), jnp.int32) for _ in range(NQ))
    for chunk in range(3):
        ebase = pt * EPT + chunk * (CH_V * 16)
        pltpu.sync_copy(src_hbm.at[pl.ds(ebase, CH_V * 16)], csrc)
        pltpu.sync_copy(dst_hbm.at[pl.ds(ebase, CH_V * 16)], cdst)
        pltpu.sync_copy(ea_hbm.at[pl.ds(ebase, CH_V * 16)], cea)

        def step(v, offs_c):
            d = cdst[pl.ds(v * 16, 16)]
            sv = csrc[pl.ds(v * 16, 16)]
            av = cea[pl.ds(v * 16, 16)]
            q = (d >= R).astype(jnp.int32)
            for kk in range(2, NQ):
                q = q + (d >= kk * R).astype(jnp.int32)
            rel = d - q * R
            new = []
            for b in range(NQ):
                m = q == b
                prefix = plsc.cumsum(m.astype(jnp.int32))
                pos = offs_c[b] + prefix - 1
                bvec = jnp.full((16,), b, jnp.int32)
                plsc.store_scatter(ssrc, [bvec, pos], sv, mask=m)
                plsc.store_scatter(sea, [bvec, pos], av, mask=m)
                plsc.store_scatter(srel, [bvec, pos], rel, mask=m)
                new.append(offs_c[b] + plsc.all_reduce_population_count(m))
            return tuple(new)
        offs = lax.fori_loop(0, CH_V, step, offs)

    for b in range(NQ):
        dsto = b * CAP + pt * CPT
        pltpu.sync_copy(ssrc.at[b], bsrc_hbm.at[pl.ds(dsto, CPT)])
        pltpu.sync_copy(sea.at[b], bea_hbm.at[pl.ds(dsto, CPT)])
        pltpu.sync_copy(srel.at[b], brel_hbm.at[pl.ds(dsto, CPT)])


@functools.partial(
    pl.kernel,
    out_type=jax.ShapeDtypeStruct((NQ * R, H), jnp.float32),
    mesh=_mesh,
    compiler_params=pltpu.CompilerParams(needs_layout_passes=False),
    scratch_types=[
        pltpu.VMEM_SHARED((ACC_ROWS, H), jnp.float32),
        pltpu.VMEM((ZR, H), jnp.float32),
        pltpu.VMEM((2, H), jnp.float32),
        pltpu.VMEM((SPT,), jnp.int32),
        pltpu.VMEM((128, H), jnp.float32),
        pltpu.VMEM((128, H), jnp.float32),
        pltpu.VMEM((128, H), jnp.float32),
        pltpu.VMEM((128,), jnp.int32),
        pltpu.VMEM((128,), jnp.int32),
        pltpu.VMEM((128,), jnp.int32),
        pltpu.VMEM((128,), jnp.float32),
        pltpu.VMEM((128,), jnp.float32),
        pltpu.VMEM((128,), jnp.float32),
        pltpu.SemaphoreType.DMA,
        pltpu.SemaphoreType.DMA,
        pltpu.SemaphoreType.DMA,
        pltpu.SemaphoreType.DMA,
        pltpu.SemaphoreType.DMA,
        pltpu.SemaphoreType.DMA,
        pltpu.SemaphoreType.DMA,
    ],
)
def _msg_k(tab_hbm, bsrc_hbm, bea_hbm, brel_hbm, webe_hbm, out_hbm,
           acc, zbuf, webe, isrc, rows0, rows1, rows2,
           vrel0, vrel1, vrel2, vea0, vea1, vea2,
           gs0, gs1, gs2, ss0, ss1, ss2, zsem):
    c = lax.axis_index("c")
    s = lax.axis_index("s")
    ROWS = (rows0, rows1, rows2)
    VREL = (vrel0, vrel1, vrel2)
    VEA = (vea0, vea1, vea2)
    GS = (gs0, gs1, gs2)
    SS = (ss0, ss1, ss2)
    pltpu.sync_copy(webe_hbm, webe)

    def zfill(i, carry):
        zero = jnp.zeros((16,), jnp.float32)
        for k in range(H // 16):
            zbuf[i, pl.ds(k * 16, 16)] = zero
        return carry
    lax.fori_loop(0, ZR, zfill, 0)

    wek = [webe[0, pl.ds(k * 16, 16)] for k in range(H // 16)]
    bek = [webe[1, pl.ds(k * 16, 16)] for k in range(H // 16)]

    def swait(b):
        pltpu.make_async_copy(tab_hbm.at[pl.ds(0, 128)], ROWS[b],
                              SS[b]).wait()

    def compute(b):
        rows = ROWS[b]
        veab = VEA[b]

        def grp(jj, carry2):
            ea16 = veab[pl.ds(jj * 16, 16)]
            for l in range(16):
                j = jj * 16 + l
                eab = ea16.at[jnp.full((16,), l, jnp.int32)].get(
                    mode="promise_in_bounds")
                for k in range(H // 16):
                    rv = rows[j, pl.ds(k * 16, 16)]
                    rows[j, pl.ds(k * 16, 16)] = jnp.maximum(
                        rv + eab * wek[k] + bek[k], 0.0)
            return carry2
        lax.fori_loop(0, 8, grp, 0)

    NIT = NBLK // 3

    def pass_body(p, carry):
        q = 2 * p + c
        zr0 = s * (ACC_ROWS // 16)
        NZC = ACC_ROWS // 16 // ZR
        for i in range(NZC):
            pltpu.async_copy(zbuf, acc.at[pl.ds(zr0 + i * ZR, ZR)], zsem)
        for i in range(NZC):
            pltpu.make_async_copy(tab_hbm.at[pl.ds(0, ZR)], zbuf,
                                  zsem).wait()
        plsc.subcore_barrier()

        off = pl.multiple_of(q * CAP + s * SPT, SPT)
        pltpu.sync_copy(bsrc_hbm.at[pl.ds(off, SPT)], isrc)

        def gissue(i, b):
            pltpu.async_copy(tab_hbm.at[isrc.at[pl.ds(i * 128, 128)]],
                             ROWS[b], GS[b])
            pltpu.async_copy(bea_hbm.at[pl.ds(off + i * 128, 128)],
                             VEA[b], GS[b])
            pltpu.async_copy(brel_hbm.at[pl.ds(off + i * 128, 128)],
                             VREL[b], GS[b])

        def gwait(b):
            pltpu.make_async_copy(tab_hbm.at[pl.ds(0, 128)], ROWS[b],
                                  GS[b]).wait()
            pltpu.make_async_copy(bea_hbm.at[pl.ds(0, 128)], VEA[b],
                                  GS[b]).wait()
            pltpu.make_async_copy(brel_hbm.at[pl.ds(0, 128)], VREL[b],
                                  GS[b]).wait()

        def sissue(b):
            pltpu.async_copy(ROWS[b], acc.at[VREL[b]], SS[b], add=True)

        gissue(0, 0)
        gissue(1, 1)

        def it_body(it, carry):
            b0 = 3 * it
            gwait(0)
            compute(0)
            sissue(0)
            @pl.when(it > 0)
            def _():
                swait(2)
            gissue(b0 + 2, 2)
            gwait(1)
            compute(1)
            sissue(1)

            @pl.when(it < NIT - 1)
            def _():
                swait(0)
                gissue(b0 + 3, 0)
            gwait(2)
            compute(2)
            sissue(2)

            @pl.when(it < NIT - 1)
            def _():
                swait(1)
                gissue(b0 + 4, 1)
            return carry
        lax.fori_loop(0, NIT, it_body, 0)
        swait(0)
        swait(1)
        swait(2)

        plsc.subcore_barrier()
        orow = pl.multiple_of(q * R + s * CPR, CPR)
        pltpu.sync_copy(acc.at[pl.ds(s * CPR, CPR)],
                        out_hbm.at[pl.ds(orow, CPR)])
        plsc.subcore_barrier()
        return carry

    lax.fori_loop(0, NQ // 2, pass_body, 0)


@functools.partial(
    pl.kernel,
    out_type=jax.ShapeDtypeStruct((NQ * R, H), jnp.float32),
    mesh=_mesh,
    compiler_params=pltpu.CompilerParams(needs_layout_passes=False),
    scratch_types=[
        pltpu.VMEM_SHARED((ACC_ROWS, H), jnp.float32),
        pltpu.VMEM((ZR, H), jnp.float32),
        pltpu.VMEM((4, H), jnp.float32),
        pltpu.VMEM((SPT,), jnp.int32),
        pltpu.VMEM((128, H), jnp.float32),
        pltpu.VMEM((128, H), jnp.float32),
        pltpu.VMEM((128, H), jnp.float32),
        pltpu.VMEM((128,), jnp.int32),
        pltpu.VMEM((128,), jnp.int32),
        pltpu.VMEM((128,), jnp.int32),
        pltpu.VMEM((128,), jnp.float32),
        pltpu.VMEM((128,), jnp.float32),
        pltpu.VMEM((128,), jnp.float32),
        pltpu.VMEM((128,), jnp.float32),
        pltpu.VMEM((128,), jnp.float32),
        pltpu.VMEM((128,), jnp.float32),
        pltpu.SemaphoreType.DMA,
        pltpu.SemaphoreType.DMA,
        pltpu.SemaphoreType.DMA,
        pltpu.SemaphoreType.DMA,
        pltpu.SemaphoreType.DMA,
        pltpu.SemaphoreType.DMA,
        pltpu.SemaphoreType.DMA,
    ],
)
def _msg1_k(x_hbm, bsrc_hbm, bea_hbm, brel_hbm, wp_hbm, out_hbm,
            acc, zbuf, wp, isrc, rows0, rows1, rows2,
            vrel0, vrel1, vrel2, vea0, vea1, vea2, vax0, vax1, vax2,
            gs0, gs1, gs2, ss0, ss1, ss2, zsem):
    c = lax.axis_index("c")
    s = lax.axis_index("s")
    ROWS = (rows0, rows1, rows2)
    VREL = (vrel0, vrel1, vrel2)
    VEA = (vea0, vea1, vea2)
    VAX = (vax0, vax1, vax2)
    GS = (gs0, gs1, gs2)
    SS = (ss0, ss1, ss2)
    pltpu.sync_copy(wp_hbm, wp)

    def zfill(i, carry):
        zero = jnp.zeros((16,), jnp.float32)
        for k in range(H // 16):
            zbuf[i, pl.ds(k * 16, 16)] = zero
        return carry
    lax.fori_loop(0, ZR, zfill, 0)

    wnk = [wp[0, pl.ds(k * 16, 16)] for k in range(H // 16)]
    wek = [wp[1, pl.ds(k * 16, 16)] for k in range(H // 16)]
    bk = [wp[2, pl.ds(k * 16, 16)] for k in range(H // 16)]

    def compute(b):
        rows = ROWS[b]
        veab = VEA[b]
        vaxb = VAX[b]

        def grp(jj, carry2):
            ea16 = veab[pl.ds(jj * 16, 16)]
            ax16 = vaxb[pl.ds(jj * 16, 16)]
            for l in range(16):
                j = jj * 16 + l
                cl = jnp.full((16,), l, jnp.int32)
                eab = ea16.at[cl].get(mode="promise_in_bounds")
                axb = ax16.at[cl].get(mode="promise_in_bounds")
                for k in range(H // 16):
                    rows[j, pl.ds(k * 16, 16)] = jnp.maximum(
                        axb * wnk[k] + (eab * wek[k] + bk[k]), 0.0)
            return carry2
        lax.fori_loop(0, 8, grp, 0)

    NIT = NBLK // 3

    def pass_body(p, carry):
        q = 2 * p + c
        zr0 = s * (ACC_ROWS // 16)
        NZC = ACC_ROWS // 16 // ZR
        for i in range(NZC):
            pltpu.async_copy(zbuf, acc.at[pl.ds(zr0 + i * ZR, ZR)], zsem)
        for i in range(NZC):
            pltpu.make_async_copy(out_hbm.at[pl.ds(0, ZR)], zbuf,
                                  zsem).wait()
        plsc.subcore_barrier()

        off = pl.multiple_of(q * CAP + s * SPT, SPT)
        pltpu.sync_copy(bsrc_hbm.at[pl.ds(off, SPT)], isrc)

        def gissue(i, b):
            pltpu.async_copy(x_hbm.at[isrc.at[pl.ds(i * 128, 128)]],
                             VAX[b], GS[b])
            pltpu.async_copy(bea_hbm.at[pl.ds(off + i * 128, 128)],
                             VEA[b], GS[b])
            pltpu.async_copy(brel_hbm.at[pl.ds(off + i * 128, 128)],
                             VREL[b], GS[b])

        def gwait(b):
            pltpu.make_async_copy(x_hbm.at[pl.ds(0, 128)], VAX[b],
                                  GS[b]).wait()
            pltpu.make_async_copy(bea_hbm.at[pl.ds(0, 128)], VEA[b],
                                  GS[b]).wait()
            pltpu.make_async_copy(brel_hbm.at[pl.ds(0, 128)], VREL[b],
                                  GS[b]).wait()

        def sissue(b):
            pltpu.async_copy(ROWS[b], acc.at[VREL[b]], SS[b], add=True)

        def swait2(b):
            pltpu.make_async_copy(out_hbm.at[pl.ds(0, 128)], ROWS[b],
                                  SS[b]).wait()

        gissue(0, 0)
        gissue(1, 1)

        def it_body(it, carry2):
            b0 = 3 * it
            gwait(0)
            compute(0)
            sissue(0)

            @pl.when(it > 0)
            def _():
                swait2(2)
            gissue(b0 + 2, 2)
            gwait(1)
            compute(1)
            sissue(1)

            @pl.when(it < NIT - 1)
            def _():
                swait2(0)
                gissue(b0 + 3, 0)
            gwait(2)
            compute(2)
            sissue(2)

            @pl.when(it < NIT - 1)
            def _():
                swait2(1)
                gissue(b0 + 4, 1)
            return carry2
        lax.fori_loop(0, NIT, it_body, 0)
        swait2(0)
        swait2(1)
        swait2(2)

        plsc.subcore_barrier()
        orow = pl.multiple_of(q * R + s * CPR, CPR)
        pltpu.sync_copy(acc.at[pl.ds(s * CPR, CPR)],
                        out_hbm.at[pl.ds(orow, CPR)])
        plsc.subcore_barrier()
        return carry

    lax.fori_loop(0, NQ // 2, pass_body, 0)


def _h0_body(x_ref, w_ref, b_ref, out_ref):
    out_ref[...] = x_ref[...] * w_ref[...] + b_ref[...]


def _h0(x, W_node, b_node):
    return pl.pallas_call(
        _h0_body,
        grid=(N // BLK,),
        in_specs=[pl.BlockSpec((BLK, 1), lambda i: (i, 0)),
                  pl.BlockSpec((1, H), lambda i: (0, 0)),
                  pl.BlockSpec((1, H), lambda i: (0, 0))],
        out_specs=pl.BlockSpec((BLK, H), lambda i: (i, 0)),
        out_shape=jax.ShapeDtypeStruct((N, H), jnp.float32),
    )(x, W_node, b_node[None, :])


def _mlp_body(h_ref, agg_ref, w1_ref, b1_ref, w2_ref, b2_ref, s_ref, be_ref,
              eps_ref, out_ref):
    h = h_ref[...]
    out = (1.0 + eps_ref[0]) * h + agg_ref[...]
    t = _lrelu(jnp.dot(out, w1_ref[...], preferred_element_type=jnp.float32)
               + b1_ref[...])
    t = jnp.dot(t, w2_ref[...], preferred_element_type=jnp.float32) + b2_ref[...]
    out_ref[...] = jnp.maximum(t * s_ref[...] + be_ref[...], 0.0)


def _mlp_block(h, agg, eps, W1, b1, W2, b2, g, be):
    s = (g / jnp.sqrt(1.0 + 1e-5))[None, :]
    return pl.pallas_call(
        _mlp_body,
        grid=(N // BLK,),
        in_specs=[
            pl.BlockSpec((BLK, H), lambda i: (i, 0)),
            pl.BlockSpec((BLK, H), lambda i: (i, 0)),

            pl.BlockSpec((H, H), lambda i: (0, 0)),
            pl.BlockSpec((1, H), lambda i: (0, 0)),
            pl.BlockSpec((H, H), lambda i: (0, 0)),
            pl.BlockSpec((1, H), lambda i: (0, 0)),
            pl.BlockSpec((1, H), lambda i: (0, 0)),
            pl.BlockSpec((1, H), lambda i: (0, 0)),
            pl.BlockSpec(memory_space=pltpu.SMEM),
        ],
        out_specs=pl.BlockSpec((BLK, H), lambda i: (i, 0)),
        out_shape=jax.ShapeDtypeStruct((N, H), jnp.float32),
    )(h, agg, W1, b1[None, :], W2, b2[None, :], s, be[None, :],
      eps.reshape(1))


def _head_body(batch_ref, h_ref, wm1_ref, bm1_ref, wm2_ref, bm2_ref, out_ref,
               sums, cnts):
    i = pl.program_id(0)

    @pl.when(i == 0)
    def _():
        sums[...] = jnp.zeros_like(sums)
        cnts[...] = jnp.zeros_like(cnts)

    bvec = batch_ref[...].reshape(1, BLK)
    oh = (bvec == lax.broadcasted_iota(jnp.int32, (G, BLK), 0)
          ).astype(jnp.float32)
    sums[...] += jnp.dot(oh, h_ref[...], preferred_element_type=jnp.float32)
    cnts[...] += jnp.sum(oh, axis=1, keepdims=True)

    @pl.when(i == N // BLK - 1)
    def _():
        pooled = sums[...] / jnp.maximum(cnts[...], 1.0)
        z = _lrelu(jnp.dot(pooled, wm1_ref[...],
                           preferred_element_type=jnp.float32) + bm1_ref[...])
        o = jnp.dot(z, wm2_ref[...],
                    preferred_element_type=jnp.float32) + bm2_ref[0, 0]
        out_ref[...] = 1.0 / (1.0 + jnp.exp(-o))


def _head(batch3, h2, Wm1, bm1, Wm2, bm2):
    return pl.pallas_call(
        _head_body,
        grid=(N // BLK,),
        in_specs=[
            pl.BlockSpec((1, 1, BLK), lambda i: (i, 0, 0)),
            pl.BlockSpec((BLK, H), lambda i: (i, 0)),
            pl.BlockSpec((H, H), lambda i: (0, 0)),
            pl.BlockSpec((1, H), lambda i: (0, 0)),
            pl.BlockSpec((H, 1), lambda i: (0, 0)),
            pl.BlockSpec((1, 1), lambda i: (0, 0)),
        ],
        out_specs=pl.BlockSpec((G, 1), lambda i: (0, 0)),
        out_shape=jax.ShapeDtypeStruct((G, 1), jnp.float32),
        scratch_shapes=[pltpu.VMEM((G, H), jnp.float32),
                        pltpu.VMEM((G, H), jnp.float32)],
    )(batch3, h2, Wm1, bm1[None, :], Wm2, bm2[None, :])


def kernel(x, edge_index, edge_attr, batch, W_node, b_node, W_edge, b_edge,
           eps1, W11, b11, W12, b12, g1, be1, eps2, W21, b21, W22, b22, g2,
           be2, Wm1, bm1, Wm2, bm2):
    src = edge_index[0]
    dst = edge_index[1]
    pad = E_PAD - E
    srcp = jnp.concatenate(
        [src, (jnp.arange(pad, dtype=jnp.int32) * 37) & 1023])
    dstp = jnp.concatenate([dst, jnp.full((pad,), NQ * R - 1, jnp.int32)])
    eap = jnp.concatenate([edge_attr[:, 0], jnp.zeros((pad,), jnp.float32)])
    webe = jnp.stack([W_edge[0], b_edge])

    wp = jnp.stack([W_node[0], W_edge[0], b_node + b_edge,
                    jnp.zeros((H,), jnp.float32)])
    bsrc, bea, brel = _bucket_k(srcp, dstp, eap)
    h0 = _h0(x, W_node, b_node)
    agg1 = _msg1_k(x[:, 0], bsrc, bea, brel, wp)
    h1 = _mlp_block(h0, agg1, eps1, W11, b11, W12, b12, g1, be1)
    agg2 = _msg_k(h1, bsrc, bea, brel, webe)
    h2 = _mlp_block(h1, agg2, eps2, W21, b21, W22, b22, g2, be2)

    batch3 = batch.reshape(N // BLK, 1, BLK)
    out = _head(batch3, h2, Wm1, bm1, Wm2, bm2)
    return out.reshape(G)


# speedup vs baseline: 1.0839x; 1.0839x over previous
import functools

import jax
import jax.numpy as jnp
from jax import lax
from jax.experimental import pallas as pl
from jax.experimental.pallas import tpu as pltpu
from jax.experimental.pallas import tpu_sc as plsc

N, E, G, H = 50000, 800000, 64, 128
BLK = 1000

R = 6272
NQ = 8
ACC_ROWS = 6400
DUMP = R
E_PAD = 800256
EPT = E_PAD // 32
CH_V = 521
CPT = 3456
CAP = 32 * CPT
SPT = CAP // 16
NBLK = SPT // 128
CPR = R // 16
ZR = 16

_mesh = plsc.VectorSubcoreMesh(core_axis_name="c", subcore_axis_name="s")


def _lrelu(x):
    return jnp.where(x >= 0, x, 0.01 * x)


@functools.partial(
    pl.kernel,
    out_type=(jax.ShapeDtypeStruct((NQ * CAP,), jnp.int32),
              jax.ShapeDtypeStruct((NQ * CAP,), jnp.float32),
              jax.ShapeDtypeStruct((NQ * CAP,), jnp.int32)),
    mesh=_mesh,
    compiler_params=pltpu.CompilerParams(needs_layout_passes=False),
    scratch_types=[
        pltpu.VMEM((CH_V * 16,), jnp.int32),
        pltpu.VMEM((CH_V * 16,), jnp.int32),
        pltpu.VMEM((CH_V * 16,), jnp.float32),
        pltpu.VMEM((NQ, CPT), jnp.int32),
        pltpu.VMEM((NQ, CPT), jnp.float32),
        pltpu.VMEM((NQ, CPT), jnp.int32),
    ],
)
def _bucket_k(src_hbm, dst_hbm, ea_hbm, bsrc_hbm, bea_hbm, brel_hbm,
              csrc, cdst, cea, ssrc, sea, srel):
    c = lax.axis_index("c")
    s = lax.axis_index("s")
    pt = c * 16 + s
    iota = lax.iota(jnp.int32, 16)

    def fill(v, carry):
        base = v * 16
        spread = (base + iota) & 1023
        rel = DUMP + ((base + iota) & 63)
        zero = jnp.zeros((16,), jnp.float32)
        for b in range(NQ):
            ssrc[b, pl.ds(base, 16)] = spread
            sea[b, pl.ds(base, 16)] = zero
            srel[b, pl.ds(base, 16)] = rel
        return carry
    lax.fori_loop(0, CPT // 16, fill, 0)

    offs = tuple(jnp.zeros((16,), jnp.int32) for _ in range(NQ))
    for chunk in range(3):
        ebase = pt * EPT + chunk * (CH_V * 16)
        pltpu.sync_copy(src_hbm.at[pl.ds(ebase, CH_V * 16)], csrc)
        pltpu.sync_copy(dst_hbm.at[pl.ds(ebase, CH_V * 16)], cdst)
        pltpu.sync_copy(ea_hbm.at[pl.ds(ebase, CH_V * 16)], cea)

        def step(v, offs_c):
            d = cdst[pl.ds(v * 16, 16)]
            sv = csrc[pl.ds(v * 16, 16)]
            av = cea[pl.ds(v * 16, 16)]
            q = (d >= R).astype(jnp.int32)
            for kk in range(2, NQ):
                q = q + (d >= kk * R).astype(jnp.int32)
            rel = d - q * R
            new = []
            for b in range(NQ):
                m = q == b
                prefix = plsc.cumsum(m.astype(jnp.int32))
                pos = offs_c[b] + prefix - 1
                bvec = jnp.full((16,), b, jnp.int32)
                plsc.store_scatter(ssrc, [bvec, pos], sv, mask=m)
                plsc.store_scatter(sea, [bvec, pos], av, mask=m)
                plsc.store_scatter(srel, [bvec, pos], rel, mask=m)
                new.append(offs_c[b] + plsc.all_reduce_population_count(m))
            return tuple(new)
        offs = lax.fori_loop(0, CH_V, step, offs)

    for b in range(NQ):
        dsto = b * CAP + pt * CPT
        pltpu.sync_copy(ssrc.at[b], bsrc_hbm.at[pl.ds(dsto, CPT)])
        pltpu.sync_copy(sea.at[b], bea_hbm.at[pl.ds(dsto, CPT)])
        pltpu.sync_copy(srel.at[b], brel_hbm.at[pl.ds(dsto, CPT)])


@functools.partial(
    pl.kernel,
    out_type=jax.ShapeDtypeStruct((NQ * R, H), jnp.float32),
    mesh=_mesh,
    compiler_params=pltpu.CompilerParams(needs_layout_passes=False),
    scratch_types=[
        pltpu.VMEM_SHARED((ACC_ROWS, H), jnp.float32),
        pltpu.VMEM((ZR, H), jnp.float32),
        pltpu.VMEM((2, H), jnp.float32),
        pltpu.VMEM((SPT,), jnp.int32),
        pltpu.VMEM((128, H), jnp.float32),
        pltpu.VMEM((128, H), jnp.float32),
        pltpu.VMEM((128, H), jnp.float32),
        pltpu.VMEM((128,), jnp.int32),
        pltpu.VMEM((128,), jnp.int32),
        pltpu.VMEM((128,), jnp.int32),
        pltpu.VMEM((128,), jnp.float32),
        pltpu.VMEM((128,), jnp.float32),
        pltpu.VMEM((128,), jnp.float32),
        pltpu.SemaphoreType.DMA,
        pltpu.SemaphoreType.DMA,
        pltpu.SemaphoreType.DMA,
        pltpu.SemaphoreType.DMA,
        pltpu.SemaphoreType.DMA,
        pltpu.SemaphoreType.DMA,
        pltpu.SemaphoreType.DMA,
    ],
)
def _msg_k(tab_hbm, bsrc_hbm, bea_hbm, brel_hbm, webe_hbm, out_hbm,
           acc, zbuf, webe, isrc, rows0, rows1, rows2,
           vrel0, vrel1, vrel2, vea0, vea1, vea2,
           gs0, gs1, gs2, ss0, ss1, ss2, zsem):
    c = lax.axis_index("c")
    s = lax.axis_index("s")
    ROWS = (rows0, rows1, rows2)
    VREL = (vrel0, vrel1, vrel2)
    VEA = (vea0, vea1, vea2)
    GS = (gs0, gs1, gs2)
    SS = (ss0, ss1, ss2)
    pltpu.sync_copy(webe_hbm, webe)

    def zfill(i, carry):
        zero = jnp.zeros((16,), jnp.float32)
        for k in range(H // 16):
            zbuf[i, pl.ds(k * 16, 16)] = zero
        return carry
    lax.fori_loop(0, ZR, zfill, 0)

    wek = [webe[0, pl.ds(k * 16, 16)] for k in range(H // 16)]
    bek = [webe[1, pl.ds(k * 16, 16)] for k in range(H // 16)]

    def swait(b):
        pltpu.make_async_copy(tab_hbm.at[pl.ds(0, 128)], ROWS[b],
                              SS[b]).wait()

    def compute(b):
        rows = ROWS[b]
        veab = VEA[b]

        def grp(jj, carry2):
            ea16 = veab[pl.ds(jj * 16, 16)]
            for l in range(16):
                j = jj * 16 + l
                eab = ea16.at[jnp.full((16,), l, jnp.int32)].get(
                    mode="promise_in_bounds")
                for k in range(H // 16):
                    rv = rows[j, pl.ds(k * 16, 16)]
                    rows[j, pl.ds(k * 16, 16)] = jnp.maximum(
                        rv + eab * wek[k] + bek[k], 0.0)
            return carry2
        lax.fori_loop(0, 8, grp, 0)

    NIT = NBLK // 3

    def pass_body(p, carry):
        q = 2 * p + c
        zr0 = s * (ACC_ROWS // 16)
        NZC = ACC_ROWS // 16 // ZR
        for i in range(NZC):
            pltpu.async_copy(zbuf, acc.at[pl.ds(zr0 + i * ZR, ZR)], zsem)
        for i in range(NZC):
            pltpu.make_async_copy(tab_hbm.at[pl.ds(0, ZR)], zbuf,
                                  zsem).wait()
        plsc.subcore_barrier()

        off = pl.multiple_of(q * CAP + s * SPT, SPT)
        pltpu.sync_copy(bsrc_hbm.at[pl.ds(off, SPT)], isrc)

        def gissue(i, b):
            pltpu.async_copy(tab_hbm.at[isrc.at[pl.ds(i * 128, 128)]],
                             ROWS[b], GS[b])
            pltpu.async_copy(bea_hbm.at[pl.ds(off + i * 128, 128)],
                             VEA[b], GS[b])
            pltpu.async_copy(brel_hbm.at[pl.ds(off + i * 128, 128)],
                             VREL[b], GS[b])

        def gwait(b):
            pltpu.make_async_copy(tab_hbm.at[pl.ds(0, 128)], ROWS[b],
                                  GS[b]).wait()
            pltpu.make_async_copy(bea_hbm.at[pl.ds(0, 128)], VEA[b],
                                  GS[b]).wait()
            pltpu.make_async_copy(brel_hbm.at[pl.ds(0, 128)], VREL[b],
                                  GS[b]).wait()

        def sissue(b):
            pltpu.async_copy(ROWS[b], acc.at[VREL[b]], SS[b], add=True)

        gissue(0, 0)
        gissue(1, 1)

        def it_body(it, carry):
            b0 = 3 * it
            gwait(0)
            compute(0)
            sissue(0)
            @pl.when(it > 0)
            def _():
                swait(2)
            gissue(b0 + 2, 2)
            gwait(1)
            compute(1)
            sissue(1)

            @pl.when(it < NIT - 1)
            def _():
                swait(0)
                gissue(b0 + 3, 0)
            gwait(2)
            compute(2)
            sissue(2)

            @pl.when(it < NIT - 1)
            def _():
                swait(1)
                gissue(b0 + 4, 1)
            return carry
        lax.fori_loop(0, NIT, it_body, 0)
        swait(0)
        swait(1)
        swait(2)

        plsc.subcore_barrier()
        orow = pl.multiple_of(q * R + s * CPR, CPR)
        pltpu.sync_copy(acc.at[pl.ds(s * CPR, CPR)],
                        out_hbm.at[pl.ds(orow, CPR)])
        plsc.subcore_barrier()
        return carry

    lax.fori_loop(0, NQ // 2, pass_body, 0)


@functools.partial(
    pl.kernel,
    out_type=jax.ShapeDtypeStruct((NQ * R, H), jnp.float32),
    mesh=_mesh,
    compiler_params=pltpu.CompilerParams(needs_layout_passes=False),
    scratch_types=[
        pltpu.VMEM_SHARED((ACC_ROWS, H), jnp.float32),
        pltpu.VMEM((ZR, H), jnp.float32),
        pltpu.VMEM((4, H), jnp.float32),
        pltpu.VMEM((SPT,), jnp.int32),
        pltpu.VMEM((128, H), jnp.float32),
        pltpu.VMEM((128, H), jnp.float32),
        pltpu.VMEM((128, H), jnp.float32),
        pltpu.VMEM((128,), jnp.int32),
        pltpu.VMEM((128,), jnp.int32),
        pltpu.VMEM((128,), jnp.int32),
        pltpu.VMEM((128,), jnp.float32),
        pltpu.VMEM((128,), jnp.float32),
        pltpu.VMEM((128,), jnp.float32),
        pltpu.VMEM((128,), jnp.float32),
        pltpu.VMEM((128,), jnp.float32),
        pltpu.VMEM((128,), jnp.float32),
        pltpu.SemaphoreType.DMA,
        pltpu.SemaphoreType.DMA,
        pltpu.SemaphoreType.DMA,
        pltpu.SemaphoreType.DMA,
        pltpu.SemaphoreType.DMA,
        pltpu.SemaphoreType.DMA,
        pltpu.SemaphoreType.DMA,
    ],
)
def _msg1_k(x_hbm, bsrc_hbm, bea_hbm, brel_hbm, wp_hbm, out_hbm,
            acc, zbuf, wp, isrc, rows0, rows1, rows2,
            vrel0, vrel1, vrel2, vea0, vea1, vea2, vax0, vax1, vax2,
            gs0, gs1, gs2, ss0, ss1, ss2, zsem):
    c = lax.axis_index("c")
    s = lax.axis_index("s")
    ROWS = (rows0, rows1, rows2)
    VREL = (vrel0, vrel1, vrel2)
    VEA = (vea0, vea1, vea2)
    VAX = (vax0, vax1, vax2)
    GS = (gs0, gs1, gs2)
    SS = (ss0, ss1, ss2)
    pltpu.sync_copy(wp_hbm, wp)

    def zfill(i, carry):
        zero = jnp.zeros((16,), jnp.float32)
        for k in range(H // 16):
            zbuf[i, pl.ds(k * 16, 16)] = zero
        return carry
    lax.fori_loop(0, ZR, zfill, 0)

    wnk = [wp[0, pl.ds(k * 16, 16)] for k in range(H // 16)]
    wek = [wp[1, pl.ds(k * 16, 16)] for k in range(H // 16)]
    bk = [wp[2, pl.ds(k * 16, 16)] for k in range(H // 16)]

    def compute(b):
        rows = ROWS[b]
        veab = VEA[b]
        vaxb = VAX[b]

        def grp(jj, carry2):
            ea16 = veab[pl.ds(jj * 16, 16)]
            ax16 = vaxb[pl.ds(jj * 16, 16)]
            for l in range(16):
                j = jj * 16 + l
                cl = jnp.full((16,), l, jnp.int32)
                eab = ea16.at[cl].get(mode="promise_in_bounds")
                axb = ax16.at[cl].get(mode="promise_in_bounds")
                for k in range(H // 16):
                    rows[j, pl.ds(k * 16, 16)] = jnp.maximum(
                        axb * wnk[k] + (eab * wek[k] + bk[k]), 0.0)
            return carry2
        lax.fori_loop(0, 8, grp, 0)

    NIT = NBLK // 3

    def pass_body(p, carry):
        q = 2 * p + c
        zr0 = s * (ACC_ROWS // 16)
        NZC = ACC_ROWS // 16 // ZR
        for i in range(NZC):
            pltpu.async_copy(zbuf, acc.at[pl.ds(zr0 + i * ZR, ZR)], zsem)
        for i in range(NZC):
            pltpu.make_async_copy(out_hbm.at[pl.ds(0, ZR)], zbuf,
                                  zsem).wait()
        plsc.subcore_barrier()

        off = pl.multiple_of(q * CAP + s * SPT, SPT)
        pltpu.sync_copy(bsrc_hbm.at[pl.ds(off, SPT)], isrc)

        def gissue(i, b):
            pltpu.async_copy(x_hbm.at[isrc.at[pl.ds(i * 128, 128)]],
                             VAX[b], GS[b])
            pltpu.async_copy(bea_hbm.at[pl.ds(off + i * 128, 128)],
                             VEA[b], GS[b])
            pltpu.async_copy(brel_hbm.at[pl.ds(off + i * 128, 128)],
                             VREL[b], GS[b])

        def gwait(b):
            pltpu.make_async_copy(x_hbm.at[pl.ds(0, 128)], VAX[b],
                                  GS[b]).wait()
            pltpu.make_async_copy(bea_hbm.at[pl.ds(0, 128)], VEA[b],
                                  GS[b]).wait()
            pltpu.make_async_copy(brel_hbm.at[pl.ds(0, 128)], VREL[b],
                                  GS[b]).wait()

        def sissue(b):
            pltpu.async_copy(ROWS[b], acc.at[VREL[b]], SS[b], add=True)

        def swait2(b):
            pltpu.make_async_copy(out_hbm.at[pl.ds(0, 128)], ROWS[b],
                                  SS[b]).wait()

        gissue(0, 0)
        gissue(1, 1)

        def it_body(it, carry2):
            b0 = 3 * it
            gwait(0)
            compute(0)
            sissue(0)

            @pl.when(it > 0)
            def _():
                swait2(2)
            gissue(b0 + 2, 2)
            gwait(1)
            compute(1)
            sissue(1)

            @pl.when(it < NIT - 1)
            def _():
                swait2(0)
                gissue(b0 + 3, 0)
            gwait(2)
            compute(2)
            sissue(2)

            @pl.when(it < NIT - 1)
            def _():
                swait2(1)
                gissue(b0 + 4, 1)
            return carry2
        lax.fori_loop(0, NIT, it_body, 0)
        swait2(0)
        swait2(1)
        swait2(2)

        plsc.subcore_barrier()
        orow = pl.multiple_of(q * R + s * CPR, CPR)
        pltpu.sync_copy(acc.at[pl.ds(s * CPR, CPR)],
                        out_hbm.at[pl.ds(orow, CPR)])
        plsc.subcore_barrier()
        return carry

    lax.fori_loop(0, NQ // 2, pass_body, 0)


def _h0_body(x_ref, w_ref, b_ref, out_ref):
    out_ref[...] = x_ref[...] * w_ref[...] + b_ref[...]


def _h0(x, W_node, b_node):
    return pl.pallas_call(
        _h0_body,
        grid=(N // BLK,),
        in_specs=[pl.BlockSpec((BLK, 1), lambda i: (i, 0)),
                  pl.BlockSpec((1, H), lambda i: (0, 0)),
                  pl.BlockSpec((1, H), lambda i: (0, 0))],
        out_specs=pl.BlockSpec((BLK, H), lambda i: (i, 0)),
        out_shape=jax.ShapeDtypeStruct((N, H), jnp.float32),
    )(x, W_node, b_node[None, :])


def _mlp_body(h_ref, agg_ref, w1_ref, b1_ref, w2_ref, b2_ref, s_ref, be_ref,
              eps_ref, out_ref):
    h = h_ref[...]
    out = (1.0 + eps_ref[0]) * h + agg_ref[...]
    t = _lrelu(jnp.dot(out, w1_ref[...], preferred_element_type=jnp.float32)
               + b1_ref[...])
    t = jnp.dot(t, w2_ref[...], preferred_element_type=jnp.float32) + b2_ref[...]
    out_ref[...] = jnp.maximum(t * s_ref[...] + be_ref[...], 0.0)


def _mlp_block(h, agg, eps, W1, b1, W2, b2, g, be):
    s = (g / jnp.sqrt(1.0 + 1e-5))[None, :]
    return pl.pallas_call(
        _mlp_body,
        grid=(N // BLK,),
        in_specs=[
            pl.BlockSpec((BLK, H), lambda i: (i, 0)),
            pl.BlockSpec((BLK, H), lambda i: (i, 0)),

            pl.BlockSpec((H, H), lambda i: (0, 0)),
            pl.BlockSpec((1, H), lambda i: (0, 0)),
            pl.BlockSpec((H, H), lambda i: (0, 0)),
            pl.BlockSpec((1, H), lambda i: (0, 0)),
            pl.BlockSpec((1, H), lambda i: (0, 0)),
            pl.BlockSpec((1, H), lambda i: (0, 0)),
            pl.BlockSpec(memory_space=pltpu.SMEM),
        ],
        out_specs=pl.BlockSpec((BLK, H), lambda i: (i, 0)),
        out_shape=jax.ShapeDtypeStruct((N, H), jnp.float32),
    )(h, agg, W1, b1[None, :], W2, b2[None, :], s, be[None, :],
      eps.reshape(1))


def _head_body(batch_ref, h_ref, wm1_ref, bm1_ref, wm2_ref, bm2_ref, out_ref,
               sums, cnts):
    i = pl.program_id(0)

    @pl.when(i == 0)
    def _():
        sums[...] = jnp.zeros_like(sums)
        cnts[...] = jnp.zeros_like(cnts)

    bvec = batch_ref[...].reshape(1, BLK)
    oh = (bvec == lax.broadcasted_iota(jnp.int32, (G, BLK), 0)
          ).astype(jnp.float32)
    sums[...] += jnp.dot(oh, h_ref[...], preferred_element_type=jnp.float32)
    cnts[...] += jnp.sum(oh, axis=1, keepdims=True)

    @pl.when(i == N // BLK - 1)
    def _():
        pooled = sums[...] / jnp.maximum(cnts[...], 1.0)
        z = _lrelu(jnp.dot(pooled, wm1_ref[...],
                           preferred_element_type=jnp.float32) + bm1_ref[...])
        o = jnp.dot(z, wm2_ref[...],
                    preferred_element_type=jnp.float32) + bm2_ref[0, 0]
        out_ref[...] = 1.0 / (1.0 + jnp.exp(-o))


def _head(batch3, h2, Wm1, bm1, Wm2, bm2):
    return pl.pallas_call(
        _head_body,
        grid=(N // BLK,),
        in_specs=[
            pl.BlockSpec((1, 1, BLK), lambda i: (i, 0, 0)),
            pl.BlockSpec((BLK, H), lambda i: (i, 0)),
            pl.BlockSpec((H, H), lambda i: (0, 0)),
            pl.BlockSpec((1, H), lambda i: (0, 0)),
            pl.BlockSpec((H, 1), lambda i: (0, 0)),
            pl.BlockSpec((1, 1), lambda i: (0, 0)),
        ],
        out_specs=pl.BlockSpec((G, 1), lambda i: (0, 0)),
        out_shape=jax.ShapeDtypeStruct((G, 1), jnp.float32),
        scratch_shapes=[pltpu.VMEM((G, H), jnp.float32),
                        pltpu.VMEM((G, H), jnp.float32)],
    )(batch3, h2, Wm1, bm1[None, :], Wm2, bm2[None, :])


def kernel(x, edge_index, edge_attr, batch, W_node, b_node, W_edge, b_edge,
           eps1, W11, b11, W12, b12, g1, be1, eps2, W21, b21, W22, b22, g2,
           be2, Wm1, bm1, Wm2, bm2):
    src = edge_index[0]
    dst = edge_index[1]
    pad = E_PAD - E
    srcp = jnp.concatenate(
        [src, (jnp.arange(pad, dtype=jnp.int32) * 37) & 1023])
    dstp = jnp.concatenate([dst, jnp.full((pad,), NQ * R - 1, jnp.int32)])
    eap = jnp.concatenate([edge_attr[:, 0], jnp.zeros((pad,), jnp.float32)])
    webe = jnp.stack([W_edge[0], b_edge])

    wp = jnp.stack([W_node[0], W_edge[0], b_node + b_edge,
                    jnp.zeros((H,), jnp.float32)])
    bsrc, bea, brel = _bucket_k(srcp, dstp, eap)
    h0 = _h0(x, W_node, b_node)
    agg1 = _msg_k(h0, bsrc, bea, brel, webe)
    h1 = _mlp_block(h0, agg1, eps1, W11, b11, W12, b12, g1, be1)
    agg2 = _msg_k(h1, bsrc, bea, brel, webe)
    h2 = _mlp_block(h1, agg2, eps2, W21, b21, W22, b22, g2, be2)

    batch3 = batch.reshape(N // BLK, 1, BLK)
    out = _head(batch3, h2, Wm1, bm1, Wm2, bm2)
    return out.reshape(G)


# speedup vs baseline: 1.1258x; 1.0387x over previous
import functools

import jax
import jax.numpy as jnp
from jax import lax
from jax.experimental import pallas as pl
from jax.experimental.pallas import tpu as pltpu
from jax.experimental.pallas import tpu_sc as plsc

N, E, G, H = 50000, 800000, 64, 128
BLK = 1000

R = 6272
NQ = 8
ACC_ROWS = 6400
DUMP = R
E_PAD = 800256
EPT = E_PAD // 32
CH_V = 521
CPT = 3456
CAP = 32 * CPT
SPT = CAP // 16
NBLK = SPT // 128
CPR = R // 16
ZR = 16

_mesh = plsc.VectorSubcoreMesh(core_axis_name="c", subcore_axis_name="s")


def _lrelu(x):
    return jnp.where(x >= 0, x, 0.01 * x)


@functools.partial(
    pl.kernel,
    out_type=(jax.ShapeDtypeStruct((NQ * CAP,), jnp.int32),
              jax.ShapeDtypeStruct((NQ * CAP,), jnp.float32),
              jax.ShapeDtypeStruct((NQ * CAP,), jnp.int32)),
    mesh=_mesh,
    compiler_params=pltpu.CompilerParams(needs_layout_passes=False),
    scratch_types=[
        pltpu.VMEM((CH_V * 16,), jnp.int32),
        pltpu.VMEM((CH_V * 16,), jnp.int32),
        pltpu.VMEM((CH_V * 16,), jnp.float32),
        pltpu.VMEM((NQ, CPT), jnp.int32),
        pltpu.VMEM((NQ, CPT), jnp.float32),
        pltpu.VMEM((NQ, CPT), jnp.int32),
    ],
)
def _bucket_k(src_hbm, dst_hbm, ea_hbm, bsrc_hbm, bea_hbm, brel_hbm,
              csrc, cdst, cea, ssrc, sea, srel):
    c = lax.axis_index("c")
    s = lax.axis_index("s")
    pt = c * 16 + s
    iota = lax.iota(jnp.int32, 16)

    def fill(v, carry):
        base = v * 16
        spread = (base + iota) & 1023
        rel = DUMP + ((base + iota) & 63)
        zero = jnp.zeros((16,), jnp.float32)
        for b in range(NQ):
            ssrc[b, pl.ds(base, 16)] = spread
            sea[b, pl.ds(base, 16)] = zero
            srel[b, pl.ds(base, 16)] = rel
        return carry
    lax.fori_loop(0, CPT // 16, fill, 0)

    offs = tuple(jnp.zeros((16,), jnp.int32) for _ in range(NQ))
    for chunk in range(3):
        ebase = pt * EPT + chunk * (CH_V * 16)
        pltpu.sync_copy(src_hbm.at[pl.ds(ebase, CH_V * 16)], csrc)
        pltpu.sync_copy(dst_hbm.at[pl.ds(ebase, CH_V * 16)], cdst)
        pltpu.sync_copy(ea_hbm.at[pl.ds(ebase, CH_V * 16)], cea)

        def step(v, offs_c):
            d = cdst[pl.ds(v * 16, 16)]
            sv = csrc[pl.ds(v * 16, 16)]
            av = cea[pl.ds(v * 16, 16)]
            q = (d >= R).astype(jnp.int32)
            for kk in range(2, NQ):
                q = q + (d >= kk * R).astype(jnp.int32)
            rel = d - q * R
            new = []
            for b in range(NQ):
                m = q == b
                prefix = plsc.cumsum(m.astype(jnp.int32))
                pos = offs_c[b] + prefix - 1
                bvec = jnp.full((16,), b, jnp.int32)
                plsc.store_scatter(ssrc, [bvec, pos], sv, mask=m)
                plsc.store_scatter(sea, [bvec, pos], av, mask=m)
                plsc.store_scatter(srel, [bvec, pos], rel, mask=m)
                new.append(offs_c[b] + plsc.all_reduce_population_count(m))
            return tuple(new)
        offs = lax.fori_loop(0, CH_V, step, offs)

    for b in range(NQ):
        dsto = b * CAP + pt * CPT
        pltpu.sync_copy(ssrc.at[b], bsrc_hbm.at[pl.ds(dsto, CPT)])
        pltpu.sync_copy(sea.at[b], bea_hbm.at[pl.ds(dsto, CPT)])
        pltpu.sync_copy(srel.at[b], brel_hbm.at[pl.ds(dsto, CPT)])


@functools.partial(
    pl.kernel,
    out_type=jax.ShapeDtypeStruct((NQ * R, H), jnp.float32),
    mesh=_mesh,
    compiler_params=pltpu.CompilerParams(needs_layout_passes=False),
    scratch_types=[
        pltpu.VMEM_SHARED((ACC_ROWS, H), jnp.float32),
        pltpu.VMEM((ZR, H), jnp.float32),
        pltpu.VMEM((2, H), jnp.float32),
        pltpu.VMEM((SPT,), jnp.int32),
        pltpu.VMEM((128, H), jnp.float32),
        pltpu.VMEM((128, H), jnp.float32),
        pltpu.VMEM((128, H), jnp.float32),
        pltpu.VMEM((128,), jnp.int32),
        pltpu.VMEM((128,), jnp.int32),
        pltpu.VMEM((128,), jnp.int32),
        pltpu.VMEM((128,), jnp.float32),
        pltpu.VMEM((128,), jnp.float32),
        pltpu.VMEM((128,), jnp.float32),
        pltpu.SemaphoreType.DMA,
        pltpu.SemaphoreType.DMA,
        pltpu.SemaphoreType.DMA,
        pltpu.SemaphoreType.DMA,
        pltpu.SemaphoreType.DMA,
        pltpu.SemaphoreType.DMA,
        pltpu.SemaphoreType.DMA,
    ],
)
def _msg_k(tab_hbm, bsrc_hbm, bea_hbm, brel_hbm, webe_hbm, out_hbm,
           acc, zbuf, webe, isrc, rows0, rows1, rows2,
           vrel0, vrel1, vrel2, vea0, vea1, vea2,
           gs0, gs1, gs2, ss0, ss1, ss2, zsem):
    c = lax.axis_index("c")
    s = lax.axis_index("s")
    ROWS = (rows0, rows1, rows2)
    VREL = (vrel0, vrel1, vrel2)
    VEA = (vea0, vea1, vea2)
    GS = (gs0, gs1, gs2)
    SS = (ss0, ss1, ss2)
    pltpu.sync_copy(webe_hbm, webe)

    def zfill(i, carry):
        zero = jnp.zeros((16,), jnp.float32)
        for k in range(H // 16):
            zbuf[i, pl.ds(k * 16, 16)] = zero
        return carry
    lax.fori_loop(0, ZR, zfill, 0)

    wek = [webe[0, pl.ds(k * 16, 16)] for k in range(H // 16)]
    bek = [webe[1, pl.ds(k * 16, 16)] for k in range(H // 16)]

    def swait(b):
        pltpu.make_async_copy(tab_hbm.at[pl.ds(0, 128)], ROWS[b],
                              SS[b]).wait()

    def compute(b):
        rows = ROWS[b]
        veab = VEA[b]

        def grp(jj, carry2):
            ea16 = veab[pl.ds(jj * 16, 16)]
            for l in range(16):
                j = jj * 16 + l
                eab = ea16.at[jnp.full((16,), l, jnp.int32)].get(
                    mode="promise_in_bounds")
                for k in range(H // 16):
                    rv = rows[j, pl.ds(k * 16, 16)]
                    rows[j, pl.ds(k * 16, 16)] = jnp.maximum(
                        rv + eab * wek[k] + bek[k], 0.0)
            return carry2
        lax.fori_loop(0, 8, grp, 0)

    NIT = NBLK // 3

    def pass_body(p, carry):
        q = 2 * p + c
        zr0 = s * (ACC_ROWS // 16)
        NZC = ACC_ROWS // 16 // ZR
        for i in range(NZC):
            pltpu.async_copy(zbuf, acc.at[pl.ds(zr0 + i * ZR, ZR)], zsem)
        for i in range(NZC):
            pltpu.make_async_copy(tab_hbm.at[pl.ds(0, ZR)], zbuf,
                                  zsem).wait()
        plsc.subcore_barrier()

        off = pl.multiple_of(q * CAP + s * SPT, SPT)
        pltpu.sync_copy(bsrc_hbm.at[pl.ds(off, SPT)], isrc)

        def gissue(i, b):
            pltpu.async_copy(tab_hbm.at[isrc.at[pl.ds(i * 128, 128)]],
                             ROWS[b], GS[b])
            pltpu.async_copy(bea_hbm.at[pl.ds(off + i * 128, 128)],
                             VEA[b], GS[b])
            pltpu.async_copy(brel_hbm.at[pl.ds(off + i * 128, 128)],
                             VREL[b], GS[b])

        def gwait(b):
            pltpu.make_async_copy(tab_hbm.at[pl.ds(0, 128)], ROWS[b],
                                  GS[b]).wait()
            pltpu.make_async_copy(bea_hbm.at[pl.ds(0, 128)], VEA[b],
                                  GS[b]).wait()
            pltpu.make_async_copy(brel_hbm.at[pl.ds(0, 128)], VREL[b],
                                  GS[b]).wait()

        def sissue(b):
            pltpu.async_copy(ROWS[b], acc.at[VREL[b]], SS[b], add=True)

        gissue(0, 0)
        gissue(1, 1)

        def it_body(it, carry):
            b0 = 3 * it
            gwait(0)
            compute(0)
            sissue(0)
            @pl.when(it > 0)
            def _():
                swait(2)
            gissue(b0 + 2, 2)
            gwait(1)
            compute(1)
            sissue(1)

            @pl.when(it < NIT - 1)
            def _():
                swait(0)
                gissue(b0 + 3, 0)
            gwait(2)
            compute(2)
            sissue(2)

            @pl.when(it < NIT - 1)
            def _():
                swait(1)
                gissue(b0 + 4, 1)
            return carry
        lax.fori_loop(0, NIT, it_body, 0)
        swait(0)
        swait(1)
        swait(2)

        plsc.subcore_barrier()
        orow = pl.multiple_of(q * R + s * CPR, CPR)
        pltpu.sync_copy(acc.at[pl.ds(s * CPR, CPR)],
                        out_hbm.at[pl.ds(orow, CPR)])
        plsc.subcore_barrier()
        return carry

    lax.fori_loop(0, NQ // 2, pass_body, 0)


def _h0_body(x_ref, w_ref, b_ref, out_ref):
    out_ref[...] = x_ref[...] * w_ref[...] + b_ref[...]


def _h0(x, W_node, b_node):
    return pl.pallas_call(
        _h0_body,
        grid=(N // BLK,),
        in_specs=[pl.BlockSpec((BLK, 1), lambda i: (i, 0)),
                  pl.BlockSpec((1, H), lambda i: (0, 0)),
                  pl.BlockSpec((1, H), lambda i: (0, 0))],
        out_specs=pl.BlockSpec((BLK, H), lambda i: (i, 0)),
        out_shape=jax.ShapeDtypeStruct((N, H), jnp.float32),
    )(x, W_node, b_node[None, :])


def _mlp_body(h_ref, agg_ref, w1_ref, b1_ref, w2_ref, b2_ref, s_ref, be_ref,
              eps_ref, out_ref):
    h = h_ref[...]
    out = (1.0 + eps_ref[0]) * h + agg_ref[...]
    t = _lrelu(jnp.dot(out, w1_ref[...], preferred_element_type=jnp.float32)
               + b1_ref[...])
    t = jnp.dot(t, w2_ref[...], preferred_element_type=jnp.float32) + b2_ref[...]
    out_ref[...] = jnp.maximum(t * s_ref[...] + be_ref[...], 0.0)


def _mlp_block(h, agg, eps, W1, b1, W2, b2, g, be):
    s = (g / jnp.sqrt(1.0 + 1e-5))[None, :]
    return pl.pallas_call(
        _mlp_body,
        grid=(N // BLK,),
        in_specs=[
            pl.BlockSpec((BLK, H), lambda i: (i, 0)),
            pl.BlockSpec((BLK, H), lambda i: (i, 0)),

            pl.BlockSpec((H, H), lambda i: (0, 0)),
            pl.BlockSpec((1, H), lambda i: (0, 0)),
            pl.BlockSpec((H, H), lambda i: (0, 0)),
            pl.BlockSpec((1, H), lambda i: (0, 0)),
            pl.BlockSpec((1, H), lambda i: (0, 0)),
            pl.BlockSpec((1, H), lambda i: (0, 0)),
            pl.BlockSpec(memory_space=pltpu.SMEM),
        ],
        out_specs=pl.BlockSpec((BLK, H), lambda i: (i, 0)),
        out_shape=jax.ShapeDtypeStruct((N, H), jnp.float32),
    )(h, agg, W1, b1[None, :], W2, b2[None, :], s, be[None, :],
      eps.reshape(1))


def _mlp2_head_body(batch_ref, h_ref, agg_ref, w1_ref, b1_ref, w2_ref,
                    b2_ref, s_ref, be_ref, wm1_ref, bm1_ref, wm2_ref,
                    bm2_ref, eps_ref, out_ref, sums, cnts):
    i = pl.program_id(0)

    @pl.when(i == 0)
    def _():
        sums[...] = jnp.zeros_like(sums)
        cnts[...] = jnp.zeros_like(cnts)

    h = h_ref[...]
    t = (1.0 + eps_ref[0]) * h + agg_ref[...]
    t = _lrelu(jnp.dot(t, w1_ref[...], preferred_element_type=jnp.float32)
               + b1_ref[...])
    t = jnp.dot(t, w2_ref[...], preferred_element_type=jnp.float32) + b2_ref[...]
    h2 = jnp.maximum(t * s_ref[...] + be_ref[...], 0.0)

    bvec = batch_ref[...].reshape(1, BLK)
    oh = (bvec == lax.broadcasted_iota(jnp.int32, (G, BLK), 0)
          ).astype(jnp.float32)
    sums[...] += jnp.dot(oh, h2, preferred_element_type=jnp.float32)
    cnts[...] += jnp.sum(oh, axis=1, keepdims=True)

    @pl.when(i == N // BLK - 1)
    def _():
        pooled = sums[...] / jnp.maximum(cnts[...], 1.0)
        z = _lrelu(jnp.dot(pooled, wm1_ref[...],
                           preferred_element_type=jnp.float32) + bm1_ref[...])
        o = jnp.dot(z, wm2_ref[...],
                    preferred_element_type=jnp.float32) + bm2_ref[0, 0]
        out_ref[...] = 1.0 / (1.0 + jnp.exp(-o))


def _mlp2_head(batch3, h1, agg2, eps, W1, b1, W2, b2, g, be,
               Wm1, bm1, Wm2, bm2):
    s = (g / jnp.sqrt(1.0 + 1e-5))[None, :]
    return pl.pallas_call(
        _mlp2_head_body,
        grid=(N // BLK,),
        in_specs=[
            pl.BlockSpec((1, 1, BLK), lambda i: (i, 0, 0)),
            pl.BlockSpec((BLK, H), lambda i: (i, 0)),
            pl.BlockSpec((BLK, H), lambda i: (i, 0)),
            pl.BlockSpec((H, H), lambda i: (0, 0)),
            pl.BlockSpec((1, H), lambda i: (0, 0)),
            pl.BlockSpec((H, H), lambda i: (0, 0)),
            pl.BlockSpec((1, H), lambda i: (0, 0)),
            pl.BlockSpec((1, H), lambda i: (0, 0)),
            pl.BlockSpec((1, H), lambda i: (0, 0)),
            pl.BlockSpec((H, H), lambda i: (0, 0)),
            pl.BlockSpec((1, H), lambda i: (0, 0)),
            pl.BlockSpec((H, 1), lambda i: (0, 0)),
            pl.BlockSpec((1, 1), lambda i: (0, 0)),
            pl.BlockSpec(memory_space=pltpu.SMEM),
        ],
        out_specs=pl.BlockSpec((G, 1), lambda i: (0, 0)),
        out_shape=jax.ShapeDtypeStruct((G, 1), jnp.float32),
        scratch_shapes=[pltpu.VMEM((G, H), jnp.float32),
                        pltpu.VMEM((G, H), jnp.float32)],
    )(batch3, h1, agg2, W1, b1[None, :], W2, b2[None, :], s, be[None, :],
      Wm1, bm1[None, :], Wm2, bm2[None, :], eps.reshape(1))


def kernel(x, edge_index, edge_attr, batch, W_node, b_node, W_edge, b_edge,
           eps1, W11, b11, W12, b12, g1, be1, eps2, W21, b21, W22, b22, g2,
           be2, Wm1, bm1, Wm2, bm2):
    src = edge_index[0]
    dst = edge_index[1]
    pad = E_PAD - E
    srcp = jnp.concatenate(
        [src, (jnp.arange(pad, dtype=jnp.int32) * 37) & 1023])
    dstp = jnp.concatenate([dst, jnp.full((pad,), NQ * R - 1, jnp.int32)])
    eap = jnp.concatenate([edge_attr[:, 0], jnp.zeros((pad,), jnp.float32)])
    webe = jnp.stack([W_edge[0], b_edge])

    bsrc, bea, brel = _bucket_k(srcp, dstp, eap)
    h0 = _h0(x, W_node, b_node)
    agg1 = _msg_k(h0, bsrc, bea, brel, webe)
    h1 = _mlp_block(h0, agg1, eps1, W11, b11, W12, b12, g1, be1)
    agg2 = _msg_k(h1, bsrc, bea, brel, webe)

    batch3 = batch.reshape(N // BLK, 1, BLK)
    out = _mlp2_head(batch3, h1, agg2, eps2, W21, b21, W22, b22, g2, be2,
                     Wm1, bm1, Wm2, bm2)
    return out.reshape(G)
